# baseline (speedup 1.0000x reference)
import functools

import jax
import jax.numpy as jnp
from jax import lax
from jax.experimental import pallas as pl
from jax.experimental.pallas import tpu as pltpu
from jax.experimental.pallas import tpu_sc as plsc

INP_SIZE = 1000000
HIDDEN = 32
W0, W1, W2 = 0.1, 0.2, 0.4
ROW = 200
RPAD = 208
SPLIT = 96
LANES = 16
N_WORKERS = 32


def _body(x_hbm, table_hbm, out_hbm, xall, idxs, rows, outb, sg0, sg1, so):
    n_rows = x_hbm.shape[0]
    rows_per_w = n_rows // N_WORKERS

    wid = lax.axis_index("s") * 2 + lax.axis_index("c")
    r0 = wid * rows_per_w

    stage = [
        pltpu.async_copy(x_hbm.at[r0 + i], xall.at[pl.ds(ROW * i, ROW)], sg0)
        for i in range(rows_per_w)
    ]
    for cp in stage:
        cp.wait()
    xall[pl.ds(ROW * rows_per_w, LANES)] = jnp.zeros((LANES,), jnp.int32)

    def build_idx(c, slot):
        @plsc.parallel_loop(0, RPAD // LANES)
        def _(v):
            t = xall[pl.ds(c * ROW + v * LANES, LANES)]
            for s in range(5):
                u = jnp.clip(t + (s - 2), 0, INP_SIZE - 1)
                idxs[slot, s, pl.ds(v * LANES, LANES)] = u

    def fire_gathers(slot, sem):
        for s in range(5):
            pltpu.async_copy(
                table_hbm.at[idxs.at[slot, s, pl.ds(0, SPLIT)]],
                rows.at[slot, s, pl.ds(0, SPLIT)],
                sem,
            )
            pltpu.async_copy(
                table_hbm.at[idxs.at[slot, s, pl.ds(SPLIT, RPAD - SPLIT)]],
                rows.at[slot, s, pl.ds(SPLIT, RPAD - SPLIT)],
                sem,
            )

    def drain_gathers(slot, sem):
        for s in range(5):
            pltpu.make_async_copy(
                table_hbm.at[idxs.at[slot, s, pl.ds(0, SPLIT)]],
                rows.at[slot, s, pl.ds(0, SPLIT)],
                sem,
            ).wait()
            pltpu.make_async_copy(
                table_hbm.at[idxs.at[slot, s, pl.ds(SPLIT, RPAD - SPLIT)]],
                rows.at[slot, s, pl.ds(SPLIT, RPAD - SPLIT)],
                sem,
            ).wait()

    def wait_out():
        pltpu.make_async_copy(
            outb.at[0, pl.ds(0, ROW)], out_hbm.at[r0], so
        ).wait()

    def compute(c, slot):
        @plsc.parallel_loop(0, RPAD, unroll=4)
        def _(j):
            for half in range(HIDDEN // LANES):
                ln = pl.ds(half * LANES, LANES)
                f0 = rows[slot, 0, j, ln]
                f1 = rows[slot, 1, j, ln]
                f2 = rows[slot, 2, j, ln]
                f3 = rows[slot, 3, j, ln]
                f4 = rows[slot, 4, j, ln]
                outb[slot, j, ln] = W0 * (f0 + f4) + W1 * (f1 + f3) + W2 * f2

        pltpu.async_copy(outb.at[slot, pl.ds(0, ROW)], out_hbm.at[r0 + c], so)

    build_idx(0, 0)
    fire_gathers(0, sg0)

    def pair_body(k, carry):
        c0 = 2 * k
        c1 = c0 + 1

        build_idx(c1, 1)
        fire_gathers(1, sg1)

        drain_gathers(0, sg0)

        @pl.when(k > 0)
        def _():
            wait_out()

        compute(c0, 0)

        @pl.when(k < rows_per_w // 2 - 1)
        def _():
            build_idx(c0 + 2, 0)
            fire_gathers(0, sg0)

        drain_gathers(1, sg1)

        @pl.when(k > 0)
        def _():
            wait_out()

        compute(c1, 1)
        return carry

    lax.fori_loop(0, rows_per_w // 2, pair_body, 0)

    wait_out()
    wait_out()


def kernel(x, table):
    b, t = x.shape
    rows_per_w = b // N_WORKERS

    mesh = plsc.VectorSubcoreMesh(core_axis_name="c", subcore_axis_name="s")
    run = functools.partial(
        pl.kernel,
        mesh=mesh,
        out_type=jax.ShapeDtypeStruct((b, t, HIDDEN), jnp.float32),
        scratch_types=[
            pltpu.VMEM((rows_per_w * ROW + LANES,), jnp.int32),
            pltpu.VMEM((2, 5, RPAD), jnp.int32),
            pltpu.VMEM((2, 5, RPAD, HIDDEN), jnp.float32),
            pltpu.VMEM((2, RPAD, HIDDEN), jnp.float32),
            pltpu.SemaphoreType.DMA,
            pltpu.SemaphoreType.DMA,
            pltpu.SemaphoreType.DMA,
        ],
        compiler_params=pltpu.CompilerParams(use_tc_tiling_on_sc=False),
    )(_body)

    return run(x.astype(jnp.int32), table)


# speedup vs baseline: 1.0008x; 1.0008x over previous
import functools

import jax
import jax.numpy as jnp
from jax import lax
from jax.experimental import pallas as pl
from jax.experimental.pallas import tpu as pltpu
from jax.experimental.pallas import tpu_sc as plsc

INP_SIZE = 1000000
HIDDEN = 32
W0, W1, W2 = 0.1, 0.2, 0.4
ROW = 200
RPAD = 208
SPLIT = 96
LANES = 16
N_WORKERS = 32


def _body(x_hbm, table_hbm, out_hbm, xall, idxs, rows, outb, sg0, sg1, so):
    n_rows = x_hbm.shape[0]
    rows_per_w = n_rows // N_WORKERS

    wid = lax.axis_index("s") * 2 + lax.axis_index("c")
    r0 = wid * rows_per_w

    stage = [
        pltpu.async_copy(x_hbm.at[r0 + i], xall.at[pl.ds(ROW * i, ROW)], sg0)
        for i in range(rows_per_w)
    ]
    for cp in stage:
        cp.wait()
    xall[pl.ds(ROW * rows_per_w, LANES)] = jnp.zeros((LANES,), jnp.float32)

    def build_idx(c, slot):
        @plsc.parallel_loop(0, RPAD // LANES)
        def _(v):
            t = xall[pl.ds(c * ROW + v * LANES, LANES)].astype(jnp.int32)
            for s in range(5):
                u = jnp.clip(t + (s - 2), 0, INP_SIZE - 1)
                idxs[slot, s, pl.ds(v * LANES, LANES)] = u

    def fire_gathers(slot, sem):
        for s in range(5):
            pltpu.async_copy(
                table_hbm.at[idxs.at[slot, s, pl.ds(0, SPLIT)]],
                rows.at[slot, s, pl.ds(0, SPLIT)],
                sem,
            )
            pltpu.async_copy(
                table_hbm.at[idxs.at[slot, s, pl.ds(SPLIT, RPAD - SPLIT)]],
                rows.at[slot, s, pl.ds(SPLIT, RPAD - SPLIT)],
                sem,
            )

    def drain_gathers(slot, sem):
        for s in range(5):
            pltpu.make_async_copy(
                table_hbm.at[idxs.at[slot, s, pl.ds(0, SPLIT)]],
                rows.at[slot, s, pl.ds(0, SPLIT)],
                sem,
            ).wait()
            pltpu.make_async_copy(
                table_hbm.at[idxs.at[slot, s, pl.ds(SPLIT, RPAD - SPLIT)]],
                rows.at[slot, s, pl.ds(SPLIT, RPAD - SPLIT)],
                sem,
            ).wait()

    def wait_out():
        pltpu.make_async_copy(
            outb.at[0, pl.ds(0, ROW)], out_hbm.at[r0], so
        ).wait()

    def compute(c, slot):
        @plsc.parallel_loop(0, RPAD, unroll=4)
        def _(j):
            for half in range(HIDDEN // LANES):
                ln = pl.ds(half * LANES, LANES)
                f0 = rows[slot, 0, j, ln]
                f1 = rows[slot, 1, j, ln]
                f2 = rows[slot, 2, j, ln]
                f3 = rows[slot, 3, j, ln]
                f4 = rows[slot, 4, j, ln]
                outb[slot, j, ln] = W0 * (f0 + f4) + W1 * (f1 + f3) + W2 * f2

        pltpu.async_copy(outb.at[slot, pl.ds(0, ROW)], out_hbm.at[r0 + c], so)

    build_idx(0, 0)
    fire_gathers(0, sg0)

    def pair_body(k, carry):
        c0 = 2 * k
        c1 = c0 + 1

        build_idx(c1, 1)
        fire_gathers(1, sg1)

        drain_gathers(0, sg0)

        @pl.when(k > 0)
        def _():
            wait_out()

        compute(c0, 0)

        @pl.when(k < rows_per_w // 2 - 1)
        def _():
            build_idx(c0 + 2, 0)
            fire_gathers(0, sg0)

        drain_gathers(1, sg1)

        @pl.when(k > 0)
        def _():
            wait_out()

        compute(c1, 1)
        return carry

    lax.fori_loop(0, rows_per_w // 2, pair_body, 0)

    wait_out()
    wait_out()


def kernel(x, table):
    b, t = x.shape
    rows_per_w = b // N_WORKERS

    mesh = plsc.VectorSubcoreMesh(core_axis_name="c", subcore_axis_name="s")
    run = functools.partial(
        pl.kernel,
        mesh=mesh,
        out_type=jax.ShapeDtypeStruct((b, t, HIDDEN), jnp.float32),
        scratch_types=[
            pltpu.VMEM((rows_per_w * ROW + LANES,), jnp.float32),
            pltpu.VMEM((2, 5, RPAD), jnp.int32),
            pltpu.VMEM((2, 5, RPAD, HIDDEN), jnp.float32),
            pltpu.VMEM((2, RPAD, HIDDEN), jnp.float32),
            pltpu.SemaphoreType.DMA,
            pltpu.SemaphoreType.DMA,
            pltpu.SemaphoreType.DMA,
        ],
        compiler_params=pltpu.CompilerParams(use_tc_tiling_on_sc=False),
    )(_body)

    return run(x.astype(jnp.float32), table)
